# baseline (speedup 1.0000x reference)
import jax
import jax.numpy as jnp
import numpy as np
from jax.experimental import pallas as pl

N = 10000
E = 160000
K = 10
PHI = 1.0


def _cheb_coe(temp, highpass):
    ct = jax.nn.relu(temp) if highpass else jax.nn.relu(temp[::-1])
    j = jnp.arange(K + 1).astype(jnp.float32)
    xj = jnp.cos((K - j + 0.5) * jnp.pi / (K + 1))
    i = jnp.arange(K + 1).astype(jnp.float32)[:, None]
    Ti = jnp.cos(i * jnp.arccos(jnp.clip(xj, -1.0, 1.0))[None, :])
    return (2.0 / (K + 1)) * (Ti @ ct)


def _cheb_prop2(x, edge_index, temp):
    n = x.shape[0]
    src = edge_index[0]
    dst = edge_index[1]
    coe_lo = _cheb_coe(temp, False)
    coe_hi = _cheb_coe(temp, True)
    deg = jax.ops.segment_sum(jnp.ones(src.shape[0], dtype=x.dtype), src, num_segments=n)
    dis = jnp.where(deg > 0, jax.lax.rsqrt(jnp.maximum(deg, 1e-12)), 0.0)
    w = -dis[src] * dis[dst]

    def mv(h):
        return jax.ops.segment_sum(h[src] * w[:, None], dst, num_segments=n)

    Tx0 = x
    Tx1 = mv(x)
    out_lo = coe_lo[0] / 2.0 * Tx0 + coe_lo[1] * Tx1
    out_hi = coe_hi[0] / 2.0 * Tx0 + coe_hi[1] * Tx1
    for k in range(2, K + 1):
        Tx2 = 2.0 * mv(Tx1) - Tx0
        out_lo = out_lo + coe_lo[k] * Tx2
        out_hi = out_hi + coe_hi[k] * Tx2
        Tx0, Tx1 = Tx1, Tx2
    return out_lo, out_hi


def _wave(h, lw, lb):
    n, d = h.shape
    hp = h.reshape(n, d // 2, 2)
    s = PHI / np.sqrt(2.0)
    ll = (hp[:, :, 0] + hp[:, :, 1]) * s
    lh = (hp[:, :, 0] - hp[:, :, 1]) * s
    return ll @ lw.T + lb, lh @ lw.T + lb


def _bil(x1, x2, W, b):
    return jnp.einsum('ni,oij,nj->no', x1, W, x2) + b


def _identity_pallas(x):
    def body(x_ref, o_ref):
        o_ref[...] = x_ref[...]
    return pl.pallas_call(
        body, out_shape=jax.ShapeDtypeStruct(x.shape, x.dtype))(x)


def kernel(feat, edge_index, feat_s0, feat_s1, edge_index_s0, edge_index_s1, shuf_feat, n_node, temp, lin_w, lin_b, wl1_lw, wl1_lb, wl2_lw, wl2_lb, d1_W, d1_b, d2_W, d2_b, d3_W, d3_b, alpha, beta):
    def enc_pair(x, ei):
        lo, hi = _cheb_prop2(x, ei, temp)
        return (jax.nn.relu(lo @ lin_w.T + lin_b),
                jax.nn.relu(hi @ lin_w.T + lin_b))

    h2, h1 = enc_pair(feat, edge_index)
    h4, h3 = enc_pair(shuf_feat, edge_index)
    h2_s0, h1_s0 = enc_pair(feat_s0, edge_index_s0)
    h2_s1, h1_s1 = enc_pair(feat_s1, edge_index_s1)

    h2 = _identity_pallas(h2)

    h2_ll, h2_lh = _wave(h2, wl1_lw, wl1_lb)
    h1_hl, h1_hh = _wave(h1, wl2_lw, wl2_lb)
    h2_mean = jax.nn.relu(jnp.mean(h2, axis=0))
    h1_mean = jax.nn.relu(jnp.mean(h1, axis=0))
    h = alpha * h2_lh + beta * h1_hh
    c = jax.nn.relu(jnp.mean(h, axis=0))
    h4_ll, h4_lh = _wave(h4, wl1_lw, wl1_lb)
    h3_ll, h3_lh = _wave(h3, wl2_lw, wl2_lb)
    s0_ll, s0_lh = _wave(h2_s0, wl1_lw, wl1_lb)
    s1_ll, s1_lh = _wave(h2_s1, wl2_lw, wl2_lb)
    s0_hl, s0_hh = _wave(h1_s0, wl1_lw, wl1_lb)
    s1_hl, s1_hh = _wave(h1_s1, wl2_lw, wl2_lb)
    cx = jnp.broadcast_to(c, h1.shape)
    sc1 = _bil(h2_lh, cx, d1_W, d1_b)[:, 0]
    sc2 = _bil(h1, cx, d1_W, d1_b)[:, 0]
    sc3 = _bil(h4_lh, cx, d1_W, d1_b)[:, 0]
    sc4 = _bil(h3, cx, d1_W, d1_b)[:, 0]
    out1 = jnp.concatenate([sc1, sc2, sc3, sc4])
    c0 = jnp.broadcast_to(h2_mean, s0_ll.shape)
    c1 = jnp.broadcast_to(h2_mean, s1_ll.shape)
    out2 = jnp.concatenate([_bil(s0_ll, c0, d2_W, d2_b)[:, 0], _bil(s1_ll, c1, d2_W, d2_b)[:, 0]])
    c0h = jnp.broadcast_to(h1_mean, s0_hl.shape)
    c1h = jnp.broadcast_to(h1_mean, s1_hl.shape)
    out3 = jnp.concatenate([_bil(s0_hl, c0h, d3_W, d3_b)[:, 0], _bil(s1_hl, c1h, d3_W, d3_b)[:, 0]])
    return (out1, out2, out3)


# speedup vs baseline: 1.1369x; 1.1369x over previous
import functools

import numpy as np
import jax
import jax.numpy as jnp
from jax import lax
from jax.experimental import pallas as pl
from jax.experimental.pallas import tpu as pltpu
from jax.experimental.pallas import tpu_sc as plsc

N = 10000
E = 160000
D = 128
K = 10
PHI = 1.0

NCHAIN = 4
NPAIR = D // 2
NW = 32
TPW = NCHAIN * NPAIR // NW
CH = 4000
MAGIC = 0x5F3759DF


def _build_sc_prop(n, e, ch):
    w2 = 2 * n
    nch = e // ch
    assert nch % 2 == 0 and ch % 16 == 0 and ch % 8 == 0
    gr = ch // 16
    mesh = plsc.VectorSubcoreMesh(core_axis_name="c", subcore_axis_name="s")

    def body(ep_hbm, xt_hbm, coe_hbm, out_hbm,
             txa, txb, acc, olo, ohi, disv, eb0, eb1, cvec, sem0, sem1):
        cid = lax.axis_index("c")
        sid = lax.axis_index("s")
        wid = sid * 2 + cid
        iot = lax.iota(jnp.int32, 16)

        chain = (wid * TPW) >> 6
        erow = jnp.maximum(chain - 1, 0)
        ebase = erow * e

        pltpu.sync_copy(coe_hbm, cvec)

        def stream_edges(process_group):
            pltpu.async_copy(ep_hbm.at[pl.ds(ebase, ch)], eb0, sem0)

            def chunk_body(buf):
                def grp(g, carry):
                    p = buf[pl.ds(g * 16, 16)]
                    process_group(p)
                    return carry
                lax.fori_loop(0, gr, grp, 0)

            def pair_body(jj, carry):
                base = ebase + jj * (2 * ch)
                pltpu.make_async_copy(
                    ep_hbm.at[pl.ds(base, ch)], eb0, sem0).wait()
                pltpu.async_copy(
                    ep_hbm.at[pl.ds(base + ch, ch)], eb1, sem1)
                chunk_body(eb0)
                pltpu.make_async_copy(
                    ep_hbm.at[pl.ds(base + ch, ch)], eb1, sem1).wait()

                @pl.when(jj < nch // 2 - 1)
                def _():
                    pltpu.async_copy(
                        ep_hbm.at[pl.ds(base + 2 * ch, ch)], eb0, sem0)

                chunk_body(eb1)
                return carry

            lax.fori_loop(0, nch // 2, pair_body, 0)

        def zacc_n(i, c):
            acc[pl.ds(i * 16, 16)] = jnp.zeros((16,), jnp.float32)
            return c
        lax.fori_loop(0, n // 16, zacc_n, 0)

        ones16 = jnp.ones((16,), jnp.float32)

        def deg_group(p):
            s = p & 0xFFFF
            plsc.addupdate_scatter(acc, [s], ones16)

        stream_edges(deg_group)

        def newton(i, c):
            d = acc[pl.ds(i * 16, 16)]
            ii = plsc.bitcast(d, jnp.int32)
            y = plsc.bitcast(MAGIC - lax.shift_right_logical(ii, 1), jnp.float32)
            y = y * (1.5 - 0.5 * d * y * y)
            y = y * (1.5 - 0.5 * d * y * y)
            y = y * (1.5 - 0.5 * d * y * y)
            disv[pl.ds(i * 16, 16)] = jnp.where(d >= 1.0, y, 0.0)
            return c
        lax.fori_loop(0, n // 16, newton, 0)

        def mv_stream(src_ref):
            def zmv(i, c):
                acc[pl.ds(i * 16, 16)] = jnp.zeros((16,), jnp.float32)
                return c
            lax.fori_loop(0, w2 // 16, zmv, 0)

            def mv_group(p):
                s = p & 0xFFFF
                d2 = lax.shift_right_logical(p, 15) & 0x1FFFE
                dv = plsc.load_gather(disv, [s])
                s2 = s + s
                x0 = plsc.load_gather(src_ref, [s2])
                plsc.addupdate_scatter(acc, [d2], x0 * dv)
                x1 = plsc.load_gather(src_ref, [s2 + 1])
                plsc.addupdate_scatter(acc, [d2 + 1], x1 * dv)

            stream_edges(mv_group)

        def task_body(t, carry):
            task = wid * TPW + t
            pair = task & (NPAIR - 1)

            pltpu.sync_copy(
                xt_hbm.at[pl.ds((chain * NPAIR + pair) * w2, w2)], txa)

            clo0 = cvec[pl.ds(0, 16)]
            chi0 = cvec[pl.ds(16, 16)]

            def init(i, c):
                x = txa[pl.ds(i * 16, 16)]
                olo[pl.ds(i * 16, 16)] = clo0 * x
                ohi[pl.ds(i * 16, 16)] = chi0 * x
                return c
            lax.fori_loop(0, w2 // 16, init, 0)

            for k in range(1, K + 1):
                src_ref, prev_ref = (txa, txb) if k % 2 == 1 else (txb, txa)
                mv_stream(src_ref)
                clo = cvec[pl.ds((2 * k) * 16, 16)]
                chi = cvec[pl.ds((2 * k + 1) * 16, 16)]

                if k == 1:
                    def upd1(i, c):
                        w0 = i * 16
                        nidx = lax.shift_right_logical(w0 + iot, 1)
                        dn = plsc.load_gather(disv, [nidx])
                        tkv = -(dn * acc[pl.ds(w0, 16)])
                        prev_ref[pl.ds(w0, 16)] = tkv
                        olo[pl.ds(w0, 16)] = olo[pl.ds(w0, 16)] + clo * tkv
                        ohi[pl.ds(w0, 16)] = ohi[pl.ds(w0, 16)] + chi * tkv
                        return c
                    lax.fori_loop(0, w2 // 16, upd1, 0)
                else:
                    def updk(i, c):
                        w0 = i * 16
                        nidx = lax.shift_right_logical(w0 + iot, 1)
                        dn = plsc.load_gather(disv, [nidx])
                        mres = -(dn * acc[pl.ds(w0, 16)])
                        tkv = 2.0 * mres - prev_ref[pl.ds(w0, 16)]
                        prev_ref[pl.ds(w0, 16)] = tkv
                        olo[pl.ds(w0, 16)] = olo[pl.ds(w0, 16)] + clo * tkv
                        ohi[pl.ds(w0, 16)] = ohi[pl.ds(w0, 16)] + chi * tkv
                        return c
                    lax.fori_loop(0, w2 // 16, updk, 0)

            base_o = (chain * NPAIR + pair) * w2
            pltpu.sync_copy(olo, out_hbm.at[pl.ds(base_o, w2)])
            pltpu.sync_copy(ohi, out_hbm.at[pl.ds(NCHAIN * NPAIR * w2 + base_o, w2)])
            return carry

        lax.fori_loop(0, TPW, task_body, 0)

    return pl.kernel(
        body,
        out_type=jax.ShapeDtypeStruct((2 * NCHAIN * NPAIR * w2,), jnp.float32),
        mesh=mesh,
        scratch_types=[
            pltpu.VMEM((w2,), jnp.float32),
            pltpu.VMEM((w2,), jnp.float32),
            pltpu.VMEM((w2,), jnp.float32),
            pltpu.VMEM((w2,), jnp.float32),
            pltpu.VMEM((w2,), jnp.float32),
            pltpu.VMEM((n,), jnp.float32),
            pltpu.VMEM((ch,), jnp.int32),
            pltpu.VMEM((ch,), jnp.int32),
            pltpu.VMEM(((K + 1) * 2 * 16,), jnp.float32),
            pltpu.SemaphoreType.DMA,
            pltpu.SemaphoreType.DMA,
        ],
        compiler_params=pltpu.CompilerParams(needs_layout_passes=False),
    )


RB = 256


def _enc_body(nreal, lh_ref, wt_ref, b_ref, x8_ref, psum_ref):
    x = lh_ref[...]
    h = jnp.maximum(
        jnp.dot(x.reshape(8 * RB, D), wt_ref[...],
                preferred_element_type=jnp.float32,
                precision=lax.Precision.HIGHEST) + b_ref[...], 0.0)
    x8_ref[...] = h.reshape(8, RB, D)
    rows = pl.program_id(0) * RB + lax.broadcasted_iota(jnp.int32, (RB, 1), 0)
    mask = rows < nreal
    h2b = jnp.where(mask, h[0:RB], 0.0)
    h1b = jnp.where(mask, h[RB:2 * RB], 0.0)
    psum_ref[...] = jnp.stack(
        [jnp.sum(h2b, axis=0), jnp.sum(h1b, axis=0)])[None]


def _bil_body(x8_ref, u_ref, bb_ref, s_ref):
    x = x8_ref[...]
    u = u_ref[...]
    s = jnp.sum(x * u[:, None, :], axis=2) + bb_ref[...][:, 0][:, None]
    s_ref[...] = s


def _cheb_coe_np(temp, highpass):
    ct = jax.nn.relu(temp) if highpass else jax.nn.relu(temp[::-1])
    j = jnp.arange(K + 1).astype(jnp.float32)
    xj = jnp.cos((K - j + 0.5) * jnp.pi / (K + 1))
    i = jnp.arange(K + 1).astype(jnp.float32)[:, None]
    Ti = jnp.cos(i * jnp.arccos(jnp.clip(xj, -1.0, 1.0))[None, :])
    return (2.0 / (K + 1)) * (Ti @ ct)


def kernel(feat, edge_index, feat_s0, feat_s1, edge_index_s0, edge_index_s1, shuf_feat, n_node, temp, lin_w, lin_b, wl1_lw, wl1_lb, wl2_lw, wl2_lb, d1_W, d1_b, d2_W, d2_b, d3_W, d3_b, alpha, beta):
    f32 = jnp.float32

    def pack(ei):
        s = ei[0].astype(jnp.int32)
        d = ei[1].astype(jnp.int32)
        return s | (d << 16)

    ep = jnp.stack([pack(edge_index), pack(edge_index_s0), pack(edge_index_s1)])

    def colpairs(x):
        return x.astype(f32).reshape(N, NPAIR, 2).transpose(1, 0, 2).reshape(NPAIR, 2 * N)

    xt = jnp.stack([colpairs(feat), colpairs(shuf_feat),
                    colpairs(feat_s0), colpairs(feat_s1)])

    coe_lo = _cheb_coe_np(temp, False)
    coe_hi = _cheb_coe_np(temp, True)
    half = jnp.concatenate([jnp.asarray([0.5], f32), jnp.ones((K,), f32)])
    coe = jnp.stack([coe_lo * half, coe_hi * half], axis=1)
    coe16 = jnp.broadcast_to(coe[:, :, None], (K + 1, 2, 16)).reshape(-1)

    sc_prop = _build_sc_prop(N, E, CH)
    out_sc = sc_prop(ep.reshape(-1), xt.reshape(-1), coe16)
    out_sc = out_sc.reshape(2, NCHAIN, NPAIR, 2 * N)

    lohi = out_sc.reshape(2, NCHAIN, NPAIR, N, 2).transpose(0, 1, 3, 2, 4)
    lohi = lohi.reshape(2, NCHAIN, N, D)
    lh = lohi.transpose(1, 0, 2, 3).reshape(8, N, D)

    npad = -(-N // RB) * RB
    lh_p = jnp.pad(lh, ((0, 0), (0, npad - N), (0, 0)))

    wt = lin_w.T.astype(f32)
    bb = lin_b.astype(f32)[None, :]

    nb = npad // RB
    x8, psums = pl.pallas_call(
        functools.partial(_enc_body, N),
        grid=(nb,),
        in_specs=[
            pl.BlockSpec((8, RB, D), lambda i: (0, i, 0)),
            pl.BlockSpec((D, D), lambda i: (0, 0)),
            pl.BlockSpec((1, D), lambda i: (0, 0)),
        ],
        out_specs=[
            pl.BlockSpec((8, RB, D), lambda i: (0, i, 0)),
            pl.BlockSpec((1, 2, D), lambda i: (i, 0, 0)),
        ],
        out_shape=[
            jax.ShapeDtypeStruct((8, npad, D), f32),
            jax.ShapeDtypeStruct((nb, 2, D), f32),
        ],
    )(lh_p, wt, bb)

    sums = jnp.sum(psums, axis=0)
    mu2 = sums[0] / N
    mu1 = sums[1] / N

    s = PHI / np.sqrt(2.0)
    a1_ll = s * jnp.repeat(wl1_lw.T, 2, axis=0)
    a2_ll = s * jnp.repeat(wl2_lw.T, 2, axis=0)
    sign = jnp.where(jnp.arange(D) % 2 == 0, 1.0, -1.0)[:, None]
    a1_lh = a1_ll * sign
    a2_lh = a2_ll * sign

    def mv_(a, b):
        return jnp.dot(a, b, precision=lax.Precision.HIGHEST)

    h2_mean = jax.nn.relu(mu2)
    h1_mean = jax.nn.relu(mu1)
    c = jax.nn.relu(alpha * (mv_(mu2, a1_lh) + wl1_lb)
                    + beta * (mv_(mu1, a2_lh) + wl2_lb))

    v1 = mv_(d1_W[0], c)
    v2 = mv_(d2_W[0], h2_mean)
    v3 = mv_(d3_W[0], h1_mean)

    u = jnp.stack([
        mv_(a1_lh, v1),
        v1,
        mv_(a1_lh, v1),
        v1,
        mv_(a1_ll, v2),
        mv_(a1_ll, v3),
        mv_(a2_ll, v2),
        mv_(a2_ll, v3),
    ])
    bias = jnp.stack([
        mv_(wl1_lb, v1) + d1_b[0],
        d1_b[0],
        mv_(wl1_lb, v1) + d1_b[0],
        d1_b[0],
        mv_(wl1_lb, v2) + d2_b[0],
        mv_(wl1_lb, v3) + d3_b[0],
        mv_(wl2_lb, v2) + d2_b[0],
        mv_(wl2_lb, v3) + d3_b[0],
    ])
    bias2 = jnp.broadcast_to(bias[:, None], (8, D))

    scores = pl.pallas_call(
        _bil_body,
        grid=(nb,),
        in_specs=[
            pl.BlockSpec((8, RB, D), lambda i: (0, i, 0)),
            pl.BlockSpec((8, D), lambda i: (0, 0)),
            pl.BlockSpec((8, D), lambda i: (0, 0)),
        ],
        out_specs=pl.BlockSpec((8, RB), lambda i: (0, i)),
        out_shape=jax.ShapeDtypeStruct((8, npad), f32),
    )(x8, u, bias2)

    scores = scores[:, :N]
    out1 = jnp.concatenate([scores[0], scores[1], scores[2], scores[3]])
    out2 = jnp.concatenate([scores[4], scores[6]])
    out3 = jnp.concatenate([scores[5], scores[7]])
    return (out1, out2, out3)
